# baseline (device time: 13916 ns/iter reference)
import jax
import jax.numpy as jnp
from jax import lax
from jax.experimental import pallas as pl
from jax.experimental.pallas import tpu as pltpu

N_DEV = 16
EPS = 1e-5


def kernel(x, gamma, beta):
    m, n_per = x.shape
    n_total = n_per * N_DEV

    def body(
        x_hbm,
        g_ref,
        b_ref,
        o_ref,
        xv_ref,
        comm_ref,
        send_sems,
        recv_sems,
        copy_sem,
    ):
        my = lax.axis_index("i")

        barrier_sem = pltpu.get_barrier_semaphore()
        for d in range(1, N_DEV):
            pl.semaphore_signal(
                barrier_sem,
                inc=1,
                device_id=(lax.rem(my + d, N_DEV),),
                device_id_type=pl.DeviceIdType.MESH,
            )

        cp = pltpu.make_async_copy(x_hbm, xv_ref, copy_sem)
        cp.start()
        cp.wait()

        xv = xv_ref[:, :]
        comm_ref[0, 0, :] = jnp.sum(xv, axis=1)
        comm_ref[0, 1, :] = jnp.sum(xv * xv, axis=1)

        pl.semaphore_wait(barrier_sem, N_DEV - 1)

        rdmas = []
        for d in range(1, N_DEV):
            peer = lax.rem(my + d, N_DEV)
            rdma = pltpu.make_async_remote_copy(
                src_ref=comm_ref.at[0],
                dst_ref=comm_ref.at[N_DEV - d],
                send_sem=send_sems.at[d],
                recv_sem=recv_sems.at[N_DEV - d],
                device_id=(peer,),
                device_id_type=pl.DeviceIdType.MESH,
            )
            rdma.start()
            rdmas.append(rdma)

        gv = g_ref[0, :][None, :]
        o_ref[:, :] = xv * gv

        for r in rdmas:
            r.wait_recv()

        total = jnp.sum(comm_ref[:, :, :], axis=0)
        mean = total[0, :] * (1.0 / n_total)
        var = total[1, :] * (1.0 / n_total) - mean * mean
        rstd = lax.rsqrt(var + EPS)
        c = mean * rstd

        bv = b_ref[0, :][None, :]
        o_ref[:, :] = o_ref[:, :] * rstd[:, None] + (bv - c[:, None] * gv)

        for r in rdmas:
            r.wait_send()

    return pl.pallas_call(
        body,
        out_shape=jax.ShapeDtypeStruct((m, n_per), jnp.float32),
        in_specs=[
            pl.BlockSpec(memory_space=pl.ANY),
            pl.BlockSpec(memory_space=pltpu.VMEM),
            pl.BlockSpec(memory_space=pltpu.VMEM),
        ],
        out_specs=pl.BlockSpec(memory_space=pltpu.VMEM),
        scratch_shapes=[
            pltpu.VMEM((m, n_per), jnp.float32),
            pltpu.VMEM((N_DEV, 2, m), jnp.float32),
            pltpu.SemaphoreType.DMA((N_DEV,)),
            pltpu.SemaphoreType.DMA((N_DEV,)),
            pltpu.SemaphoreType.DMA,
        ],
        compiler_params=pltpu.CompilerParams(collective_id=0),
    )(x, gamma.reshape(1, n_per), beta.reshape(1, n_per))
